# Initial kernel scaffold; baseline (speedup 1.0000x reference)
import jax
import jax.numpy as jnp
from jax.experimental import pallas as pl


def kernel(x, h, d_static, We1, be1, We2, be2, Wn1, bn1, Wn2, bn2, Wc1, bc1, Wc2, Wa, ba):
    raise NotImplementedError("write your pallas kernel here")


# speedup vs baseline: 1.5358x; 1.5358x over previous
import jax
import jax.numpy as jnp
import numpy as np
from jax.experimental import pallas as pl

NP = 55
NPP = 64
NF = 32
NH = 32
E = NPP * NPP
CR = 5.0


def _build_seg_matrix():
    S = np.zeros((NPP, E), np.float32)
    for i in range(NP):
        for j in range(NP):
            if i != j:
                S[i, i * NPP + j] = 1.0
    return S


_S_NP = _build_seg_matrix()


def _body(xn_ref, xt_ref, hp_ref, dsp_ref, S_ref,
          Wh1_ref, Wh2_ref, wds_ref, be1_ref, We2_ref, be2_ref,
          WaT_ref, ba_ref, Wc1_ref, bc1_ref, Wc2T_ref,
          Wn1_ref, bn1_ref, Wn2_ref, bn2_ref,
          xo_ref, ho_ref):
    h = hp_ref[0]
    x_col = xn_ref[0]
    x_row = xt_ref[0]

    dx = x_col[:, 0:1] - x_row[0:1, :]
    dy = x_col[:, 1:2] - x_row[1:2, :]
    dz = x_col[:, 2:3] - x_row[2:3, :]
    d2 = dx * dx + dy * dy + dz * dz + 1e-6
    d = jnp.sqrt(d2)

    dsb = dsp_ref[0]
    shifted = jnp.concatenate(
        [jnp.zeros((NPP, 1), jnp.float32), dsb[:, :NPP - 1]], axis=1)
    row = jax.lax.broadcasted_iota(jnp.int32, (NPP, NPP), 0)
    col = jax.lax.broadcasted_iota(jnp.int32, (NPP, NPP), 1)
    dsf = jnp.where(col < row, dsb, shifted)
    ds2 = dsf * dsf

    P = jnp.dot(h, Wh1_ref[...], preferred_element_type=jnp.float32) + be1_ref[...]
    Q = jnp.dot(h, Wh2_ref[...], preferred_element_type=jnp.float32)
    wd = wds_ref[0:1, :]
    ws = wds_ref[1:2, :]
    z = (P[:, None, :] + Q[None, :, :]
         + d2[:, :, None] * wd[None, :, :]
         + ds2[:, :, None] * ws[None, :, :])
    m1 = z * jax.nn.sigmoid(z)
    m1f = m1.reshape(E, NF)

    a = jnp.dot(m1f, We2_ref[...], preferred_element_type=jnp.float32) + be2_ref[...]
    m2 = a * jax.nn.sigmoid(a)
    att = jax.nn.sigmoid(
        jnp.sum(m2 * WaT_ref[...], axis=1, keepdims=True) + ba_ref[...])
    m3 = m2 * att

    cpre = jnp.dot(m3, Wc1_ref[...], preferred_element_type=jnp.float32) + bc1_ref[...]
    c = cpre * jax.nn.sigmoid(cpre)
    cw = jnp.tanh(jnp.sum(c * Wc2T_ref[...], axis=1, keepdims=True))
    mask = jnp.logical_and(col != row, col < NP).astype(jnp.float32)
    w2 = cw.reshape(NPP, NPP) / (d + 1.0) * mask
    ux = jnp.sum(dx * w2, axis=1, keepdims=True)
    uy = jnp.sum(dy * w2, axis=1, keepdims=True)
    uz = jnp.sum(dz * w2, axis=1, keepdims=True)
    upd = jnp.concatenate(
        [ux, uy, uz, jnp.zeros((NPP, 5), jnp.float32)], axis=1)
    xo_ref[0] = x_col + CR * upd

    m_i = jnp.dot(S_ref[...], m3, preferred_element_type=jnp.float32)
    hm = jnp.concatenate([h, m_i], axis=1)
    t = jnp.dot(hm, Wn1_ref[...], preferred_element_type=jnp.float32) + bn1_ref[...]
    t = t * jax.nn.sigmoid(t)
    hu = jnp.dot(t, Wn2_ref[...], preferred_element_type=jnp.float32) + bn2_ref[...]
    ho_ref[0] = h + hu


def kernel(x, h, d_static, We1, be1, We2, be2, Wn1, bn1, Wn2, bn2,
           Wc1, bc1, Wc2, Wa, ba):
    B = x.shape[0]
    xv = x.reshape(B, NP, 3)
    xn = jnp.pad(xv, ((0, 0), (0, NPP - NP), (0, 5)))
    xt = jnp.pad(xv.transpose(0, 2, 1), ((0, 0), (0, 5), (0, NPP - NP)))
    hp = jnp.pad(h, ((0, 0), (0, NPP - NP), (0, 0)))
    dsp = jnp.pad(d_static, ((0, 0), (0, NPP - NP), (0, NPP - (NP - 1))))
    S = jnp.asarray(_S_NP)
    Wh1 = We1[:NF]
    Wh2 = We1[NF:2 * NF]
    wds = We1[2 * NF:2 * NF + 2]

    args = (xn, xt, hp, dsp, S, Wh1, Wh2, wds,
            be1.reshape(1, NH), We2, be2.reshape(1, NF),
            Wa.T, ba.reshape(1, 1), Wc1, bc1.reshape(1, NH), Wc2.T,
            Wn1, bn1.reshape(1, NH), Wn2, bn2.reshape(1, NF))

    def batch_spec(shp):
        return pl.BlockSpec((1,) + shp, lambda b: (b, 0, 0))

    def const_spec(shp):
        return pl.BlockSpec(shp, lambda b: (0, 0))

    in_specs = [
        batch_spec((NPP, 8)), batch_spec((8, NPP)),
        batch_spec((NPP, NF)), batch_spec((NPP, NPP)),
        const_spec((NPP, E)), const_spec((NF, NH)), const_spec((NF, NH)),
        const_spec((2, NH)), const_spec((1, NH)), const_spec((NH, NF)),
        const_spec((1, NF)), const_spec((1, NF)), const_spec((1, 1)),
        const_spec((NF, NH)), const_spec((1, NH)), const_spec((1, NF)),
        const_spec((2 * NF, NH)), const_spec((1, NH)), const_spec((NH, NF)),
        const_spec((1, NF)),
    ]
    out_specs = (batch_spec((NPP, 8)), batch_spec((NPP, NF)))
    out_shape = (jax.ShapeDtypeStruct((B, NPP, 8), jnp.float32),
                 jax.ShapeDtypeStruct((B, NPP, NF), jnp.float32))

    xo, ho = pl.pallas_call(
        _body, grid=(B,), in_specs=in_specs, out_specs=out_specs,
        out_shape=out_shape)(*args)
    return xo[:, :NP, :3], ho[:, :NP, :NF]


# speedup vs baseline: 1.7774x; 1.1573x over previous
import jax
import jax.numpy as jnp
import numpy as np
from jax.experimental import pallas as pl
from jax.experimental.pallas import tpu as pltpu

NP = 55
NJ = 64
NF = 32
NH = 32
E = NP * NJ
CR = 5.0


def _build_seg_matrix():
    S = np.zeros((NP, E), np.float32)
    for i in range(NP):
        for j in range(NP):
            if i != j:
                S[i, i * NJ + j] = 1.0
    return S


_S_NP = _build_seg_matrix()


def _body(xn_ref, xt_ref, hp_ref, dsp_ref, S_ref,
          Wh1_ref, Wh2_ref, wds_ref, be1_ref, We2_ref, be2_ref,
          WaT_ref, ba_ref, Wc1_ref, bc1_ref, Wc2T_ref,
          Wn1_ref, bn1_ref, Wn2_ref, bn2_ref,
          xo_ref, ho_ref):
    h64 = hp_ref[0]
    x_col = xn_ref[0]
    x_row = xt_ref[0]

    dx = x_col[:, 0:1] - x_row[0:1, :]
    dy = x_col[:, 1:2] - x_row[1:2, :]
    dz = x_col[:, 2:3] - x_row[2:3, :]
    d2 = dx * dx + dy * dy + dz * dz + 1e-6
    d = jnp.sqrt(d2)

    dsb = dsp_ref[0]
    shifted = jnp.concatenate(
        [jnp.zeros((NP, 1), jnp.float32), dsb[:, :NJ - 1]], axis=1)
    row = jax.lax.broadcasted_iota(jnp.int32, (NP, NJ), 0)
    col = jax.lax.broadcasted_iota(jnp.int32, (NP, NJ), 1)
    dsf = jnp.where(col < row, dsb, shifted)
    ds2 = dsf * dsf

    P = jnp.dot(h64, Wh1_ref[...], preferred_element_type=jnp.float32) + be1_ref[...]
    Q = jnp.dot(h64, Wh2_ref[...], preferred_element_type=jnp.float32)
    wd = wds_ref[0:1, :]
    ws = wds_ref[1:2, :]
    z = (P[:NP, None, :] + Q[None, :, :]
         + d2[:, :, None] * wd[None, :, :]
         + ds2[:, :, None] * ws[None, :, :])
    m1 = z * jax.nn.sigmoid(z)
    m1f = m1.reshape(E, NF)

    a = jnp.dot(m1f, We2_ref[...], preferred_element_type=jnp.float32) + be2_ref[...]
    m2 = a * jax.nn.sigmoid(a)
    att = jax.nn.sigmoid(
        jnp.sum(m2 * WaT_ref[...], axis=1, keepdims=True) + ba_ref[...])
    m3 = m2 * att

    cpre = jnp.dot(m3, Wc1_ref[...], preferred_element_type=jnp.float32) + bc1_ref[...]
    c = cpre * jax.nn.sigmoid(cpre)
    c3 = c.reshape(NP, NJ, NF)
    cw = jnp.tanh(jnp.sum(c3 * Wc2T_ref[...][None, :, :], axis=-1))
    mask = jnp.logical_and(col != row, col < NP).astype(jnp.float32)
    w2 = cw / (d + 1.0) * mask
    ux = jnp.sum(dx * w2, axis=1, keepdims=True)
    uy = jnp.sum(dy * w2, axis=1, keepdims=True)
    uz = jnp.sum(dz * w2, axis=1, keepdims=True)
    upd = jnp.concatenate(
        [ux, uy, uz, jnp.zeros((NP, 5), jnp.float32)], axis=1)
    xo_ref[0] = x_col + CR * upd

    m_i = jnp.dot(S_ref[...], m3, preferred_element_type=jnp.float32)
    hm = jnp.concatenate([h64[:NP], m_i], axis=1)
    t = jnp.dot(hm, Wn1_ref[...], preferred_element_type=jnp.float32) + bn1_ref[...]
    t = t * jax.nn.sigmoid(t)
    hu = jnp.dot(t, Wn2_ref[...], preferred_element_type=jnp.float32) + bn2_ref[...]
    ho_ref[0] = h64[:NP] + hu


def kernel(x, h, d_static, We1, be1, We2, be2, Wn1, bn1, Wn2, bn2,
           Wc1, bc1, Wc2, Wa, ba):
    B = x.shape[0]
    xv = x.reshape(B, NP, 3)
    xn = jnp.pad(xv, ((0, 0), (0, 0), (0, 5)))
    xt = jnp.pad(xv.transpose(0, 2, 1), ((0, 0), (0, 5), (0, NJ - NP)))
    hp = jnp.pad(h, ((0, 0), (0, NJ - NP), (0, 0)))
    dsp = jnp.pad(d_static, ((0, 0), (0, 0), (0, NJ - (NP - 1))))
    S = jnp.asarray(_S_NP)
    Wh1 = We1[:NF]
    Wh2 = We1[NF:2 * NF]
    wds = We1[2 * NF:2 * NF + 2]

    args = (xn, xt, hp, dsp, S, Wh1, Wh2, wds,
            be1.reshape(1, NH), We2, be2.reshape(1, NF),
            Wa.T, ba.reshape(1, 1), Wc1, bc1.reshape(1, NH), Wc2.T,
            Wn1, bn1.reshape(1, NH), Wn2, bn2.reshape(1, NF))

    def batch_spec(shp):
        return pl.BlockSpec((1,) + shp, lambda b: (b, 0, 0))

    def const_spec(shp):
        return pl.BlockSpec(shp, lambda b: (0, 0))

    in_specs = [
        batch_spec((NP, 8)), batch_spec((8, NJ)),
        batch_spec((NJ, NF)), batch_spec((NP, NJ)),
        const_spec((NP, E)), const_spec((NF, NH)), const_spec((NF, NH)),
        const_spec((2, NH)), const_spec((1, NH)), const_spec((NH, NF)),
        const_spec((1, NF)), const_spec((1, NF)), const_spec((1, 1)),
        const_spec((NF, NH)), const_spec((1, NH)), const_spec((1, NF)),
        const_spec((2 * NF, NH)), const_spec((1, NH)), const_spec((NH, NF)),
        const_spec((1, NF)),
    ]
    out_specs = (batch_spec((NP, 8)), batch_spec((NP, NF)))
    out_shape = (jax.ShapeDtypeStruct((B, NP, 8), jnp.float32),
                 jax.ShapeDtypeStruct((B, NP, NF), jnp.float32))

    xo, ho = pl.pallas_call(
        _body, grid=(B,), in_specs=in_specs, out_specs=out_specs,
        out_shape=out_shape,
        compiler_params=pltpu.CompilerParams(
            dimension_semantics=("parallel",)))(*args)
    return xo[:, :, :3], ho


# speedup vs baseline: 2.1142x; 1.1895x over previous
import jax
import jax.numpy as jnp
import numpy as np
from jax.experimental import pallas as pl
from jax.experimental.pallas import tpu as pltpu

NP = 55
NJ = 64
NQ = NJ // 4
RP = NP * NQ
NF = 32
NH = 32
CR = 5.0
BF = jnp.bfloat16
F32 = jnp.float32


def _consts():
    R16 = np.zeros((RP, NJ), np.float32)
    R55 = np.tile(np.eye(NQ, dtype=np.float32), (NP, 1))
    S2 = np.kron(np.eye(NP, dtype=np.float32), np.ones((1, NQ), np.float32))
    maskp = np.zeros((RP, 4 * NF), np.float32)
    wmask4 = np.zeros((RP, 4), np.float32)
    selmask = np.zeros((RP, 4), np.float32)
    for i in range(NP):
        for q in range(NQ):
            r = i * NQ + q
            R16[r, i] = 1.0
            for g in range(4):
                j = 4 * q + g
                valid = (j != i) and (j < NP)
                if valid:
                    maskp[r, NF * g:NF * (g + 1)] = 1.0
                    wmask4[r, g] = 1.0
                if j < i:
                    selmask[r, g] = 1.0
    F16 = np.zeros((16, 8), np.float32)
    for d in range(3):
        for g in range(4):
            F16[4 * d + g, d] = 1.0
    return R16, R55, S2, maskp, wmask4, selmask, F16


_R16, _R55, _S2, _MASKP, _WMASK4, _SELMASK, _F16 = _consts()


def _body(xn_ref, xjB_ref, hp_ref, hB_ref, dsbP_ref, dsSP_ref,
          R16_ref, R16f_ref, R55_ref, S2_ref, S2f_ref, maskp_ref,
          wmask4_ref, selmask_ref, F16_ref,
          Wh1t_ref, Wh2b_ref, wdb_ref, wsb_ref, be1t_ref,
          We2b_ref, be2t_ref, Wab_ref, ba_ref, G4_ref,
          Wc1b_ref, bc1t_ref, Wc2b_ref, F4_ref,
          Wn1_ref, bn1_ref, Wn2_ref, bn2_ref,
          xo_ref, ho_ref):
    f32 = lambda a, b: jnp.dot(a, b, preferred_element_type=F32)
    bdot = lambda a, b: jnp.dot(a.astype(BF), b, preferred_element_type=F32)

    h64 = hp_ref[0]
    hB = hB_ref[0]
    xn = xn_ref[0]

    xiP = f32(R16f_ref[...], xn)
    xjP = f32(R55_ref[...], xjB_ref[0])
    dx = xiP[:, 0:1] - xjP[:, 0:4]
    dy = xiP[:, 1:2] - xjP[:, 4:8]
    dz = xiP[:, 2:3] - xjP[:, 8:12]
    d2 = dx * dx + dy * dy + dz * dz + 1e-6
    d = jnp.sqrt(d2)

    sel = selmask_ref[...]
    dsf = dsSP_ref[0] + sel * (dsbP_ref[0] - dsSP_ref[0])
    ds2 = dsf * dsf

    Ptile = bdot(h64, Wh1t_ref[...])
    Ppack = jnp.dot(R16_ref[...], Ptile.astype(BF),
                    preferred_element_type=F32)
    QB = bdot(hB, Wh2b_ref[...])
    Qpack = jnp.dot(R55_ref[...].astype(BF), QB.astype(BF),
                    preferred_element_type=F32)
    d2t = bdot(d2, wdb_ref[...])
    ds2t = bdot(ds2, wsb_ref[...])
    z = Ppack + Qpack + d2t + ds2t + be1t_ref[...]
    m1 = z * jax.nn.sigmoid(z)

    a = bdot(m1, We2b_ref[...]) + be2t_ref[...]
    m2 = a * jax.nn.sigmoid(a)
    att = jax.nn.sigmoid(bdot(m2, Wab_ref[...]) + ba_ref[...])
    attb = bdot(att, G4_ref[...])
    m3 = m2 * attb

    cpre = bdot(m3, Wc1b_ref[...]) + bc1t_ref[...]
    c = cpre * jax.nn.sigmoid(cpre)
    cw = jnp.tanh(bdot(c, Wc2b_ref[...]))
    w = cw / (d + 1.0) * wmask4_ref[...]
    prod = jnp.concatenate([dx * w, dy * w, dz * w, jnp.zeros((RP, 4), F32)],
                           axis=1)
    U = f32(S2f_ref[...], prod)
    upd = f32(U, F16_ref[...])
    xo_ref[0] = xn[:NP] + CR * upd

    m3m = m3 * maskp_ref[...]
    mi128 = jnp.dot(S2_ref[...], m3m.astype(BF),
                    preferred_element_type=F32)
    m_i = bdot(mi128, F4_ref[...])
    hm = jnp.concatenate([h64[:NP], m_i], axis=1)
    t = bdot(hm, Wn1_ref[...]) + bn1_ref[...]
    t = t * jax.nn.sigmoid(t)
    hu = bdot(t, Wn2_ref[...]) + bn2_ref[...]
    ho_ref[0] = h64[:NP] + hu


def kernel(x, h, d_static, We1, be1, We2, be2, Wn1, bn1, Wn2, bn2,
           Wc1, bc1, Wc2, Wa, ba):
    B = x.shape[0]
    xv = x.reshape(B, NP, 3)
    xn = jnp.pad(xv, ((0, 0), (0, NJ - NP), (0, 5)))
    xjB = jnp.pad(
        jnp.pad(xv, ((0, 0), (0, NJ - NP), (0, 0)))
        .reshape(B, NQ, 4, 3).transpose(0, 1, 3, 2).reshape(B, NQ, 12),
        ((0, 0), (0, 0), (0, 4)))
    hp = jnp.pad(h, ((0, 0), (0, NJ - NP), (0, 0)))
    hB = hp.reshape(B, NQ, 4, NF).reshape(B, NQ, 4 * NF)
    dsp = jnp.pad(d_static, ((0, 0), (0, 0), (0, NJ - (NP - 1))))
    dss = jnp.concatenate(
        [jnp.zeros((B, NP, 1), F32), dsp[:, :, :NJ - 1]], axis=2)
    dsbP = dsp.reshape(B, RP, 4)
    dsSP = dss.reshape(B, RP, 4)

    eye4 = jnp.eye(4, dtype=F32)
    Wh1 = We1[:NF]
    Wh2 = We1[NF:2 * NF]
    wd = We1[2 * NF:2 * NF + 1]
    ws = We1[2 * NF + 1:2 * NF + 2]
    consts = dict(
        R16=jnp.asarray(_R16, BF), R16f=jnp.asarray(_R16),
        R55=jnp.asarray(_R55), S2=jnp.asarray(_S2, BF),
        S2f=jnp.asarray(_S2), maskp=jnp.asarray(_MASKP),
        wmask4=jnp.asarray(_WMASK4), selmask=jnp.asarray(_SELMASK),
        F16=jnp.asarray(_F16),
        Wh1t=jnp.tile(Wh1, (1, 4)).astype(BF),
        Wh2b=jnp.kron(eye4, Wh2).astype(BF),
        wdb=jnp.kron(eye4, wd).astype(BF),
        wsb=jnp.kron(eye4, ws).astype(BF),
        be1t=jnp.tile(be1.reshape(1, NH), (1, 4)),
        We2b=jnp.kron(eye4, We2).astype(BF),
        be2t=jnp.tile(be2.reshape(1, NF), (1, 4)),
        Wab=jnp.kron(eye4, Wa).astype(BF),
        ba=ba.reshape(1, 1),
        G4=jnp.kron(eye4, jnp.ones((1, NF), F32)).astype(BF),
        Wc1b=jnp.kron(eye4, Wc1).astype(BF),
        bc1t=jnp.tile(bc1.reshape(1, NH), (1, 4)),
        Wc2b=jnp.kron(eye4, Wc2).astype(BF),
        F4=jnp.tile(jnp.eye(NF, dtype=F32), (4, 1)).astype(BF),
        Wn1=Wn1.astype(BF), bn1=bn1.reshape(1, NH),
        Wn2=Wn2.astype(BF), bn2=bn2.reshape(1, NF),
    )

    def batch_spec(shp):
        return pl.BlockSpec((1,) + shp, lambda b: (b, 0, 0))

    def const_spec(arr):
        return pl.BlockSpec(arr.shape, lambda b: (0, 0))

    batch_args = (xn, xjB, hp, hB, dsbP, dsSP)
    batch_shapes = ((NJ, 8), (NQ, 16), (NJ, NF), (NQ, 4 * NF), (RP, 4), (RP, 4))
    const_args = tuple(consts.values())

    in_specs = ([batch_spec(s) for s in batch_shapes]
                + [const_spec(a) for a in const_args])
    out_specs = (batch_spec((NP, 8)), batch_spec((NP, NF)))
    out_shape = (jax.ShapeDtypeStruct((B, NP, 8), F32),
                 jax.ShapeDtypeStruct((B, NP, NF), F32))

    xo, ho = pl.pallas_call(
        _body, grid=(B,), in_specs=in_specs, out_specs=out_specs,
        out_shape=out_shape,
        compiler_params=pltpu.CompilerParams(
            dimension_semantics=("parallel",)))(*batch_args, *const_args)
    return xo[:, :, :3], ho


# speedup vs baseline: 2.1945x; 1.0380x over previous
import jax
import jax.numpy as jnp
import numpy as np
from jax.experimental import pallas as pl
from jax.experimental.pallas import tpu as pltpu

NP = 55
NJ = 64
NQ = NJ // 4
RP = NP * NQ
NF = 32
NH = 32
CR = 5.0
BF = jnp.bfloat16
F32 = jnp.float32


def _consts():
    R16 = np.zeros((RP, NJ), np.float32)
    R55 = np.tile(np.eye(NQ, dtype=np.float32), (NP, 1))
    S2 = np.kron(np.eye(NP, dtype=np.float32), np.ones((1, NQ), np.float32))
    maskp = np.zeros((RP, 4 * NF), np.float32)
    wmask4 = np.zeros((RP, 4), np.float32)
    selmask = np.zeros((RP, 4), np.float32)
    for i in range(NP):
        for q in range(NQ):
            r = i * NQ + q
            R16[r, i] = 1.0
            for g in range(4):
                j = 4 * q + g
                valid = (j != i) and (j < NP)
                if valid:
                    maskp[r, NF * g:NF * (g + 1)] = 1.0
                    wmask4[r, g] = 1.0
                if j < i:
                    selmask[r, g] = 1.0
    F16 = np.zeros((16, 8), np.float32)
    for d in range(3):
        for g in range(4):
            F16[4 * d + g, d] = 1.0
    return R16, R55, S2, maskp, wmask4, selmask, F16


_R16, _R55, _S2, _MASKP, _WMASK4, _SELMASK, _F16 = _consts()


BB = 2


def _body(xn_ref, xjB_ref, hp_ref, hB_ref, dsbP_ref, dsSP_ref,
          R16_ref, R16f_ref, R55_ref, S2_ref, S2f_ref, maskp_ref,
          wmask4_ref, selmask_ref, F16_ref,
          Wh1t_ref, Wh2b_ref, wdb_ref, wsb_ref, be1t_ref,
          We2b_ref, be2t_ref, Wab_ref, ba_ref, G4_ref,
          Wc1b_ref, bc1t_ref, Wc2b_ref, F4_ref,
          Wn1_ref, bn1_ref, Wn2_ref, bn2_ref,
          xo_ref, ho_ref):
    for bb in range(BB):
        _one(bb, xn_ref, xjB_ref, hp_ref, hB_ref, dsbP_ref, dsSP_ref,
             R16_ref, R16f_ref, R55_ref, S2_ref, S2f_ref, maskp_ref,
             wmask4_ref, selmask_ref, F16_ref,
             Wh1t_ref, Wh2b_ref, wdb_ref, wsb_ref, be1t_ref,
             We2b_ref, be2t_ref, Wab_ref, ba_ref, G4_ref,
             Wc1b_ref, bc1t_ref, Wc2b_ref, F4_ref,
             Wn1_ref, bn1_ref, Wn2_ref, bn2_ref,
             xo_ref, ho_ref)


def _one(bb, xn_ref, xjB_ref, hp_ref, hB_ref, dsbP_ref, dsSP_ref,
         R16_ref, R16f_ref, R55_ref, S2_ref, S2f_ref, maskp_ref,
         wmask4_ref, selmask_ref, F16_ref,
         Wh1t_ref, Wh2b_ref, wdb_ref, wsb_ref, be1t_ref,
         We2b_ref, be2t_ref, Wab_ref, ba_ref, G4_ref,
         Wc1b_ref, bc1t_ref, Wc2b_ref, F4_ref,
         Wn1_ref, bn1_ref, Wn2_ref, bn2_ref,
         xo_ref, ho_ref):
    f32 = lambda a, b: jnp.dot(a, b, preferred_element_type=F32)
    bdot = lambda a, b: jnp.dot(a.astype(BF), b, preferred_element_type=F32)

    h64 = hp_ref[bb]
    hB = hB_ref[bb]
    xn = xn_ref[bb]

    xiP = f32(R16f_ref[...], xn)
    xjP = f32(R55_ref[...], xjB_ref[bb])
    dx = xiP[:, 0:1] - xjP[:, 0:4]
    dy = xiP[:, 1:2] - xjP[:, 4:8]
    dz = xiP[:, 2:3] - xjP[:, 8:12]
    d2 = dx * dx + dy * dy + dz * dz + 1e-6
    d = jnp.sqrt(d2)

    sel = selmask_ref[...]
    dsf = dsSP_ref[bb] + sel * (dsbP_ref[bb] - dsSP_ref[bb])
    ds2 = dsf * dsf

    Ptile = bdot(h64, Wh1t_ref[...])
    Ppack = jnp.dot(R16_ref[...], Ptile.astype(BF),
                    preferred_element_type=F32)
    QB = bdot(hB, Wh2b_ref[...])
    Qpack = jnp.dot(R55_ref[...].astype(BF), QB.astype(BF),
                    preferred_element_type=F32)
    d2t = bdot(d2, wdb_ref[...])
    ds2t = bdot(ds2, wsb_ref[...])
    z = Ppack + Qpack + d2t + ds2t + be1t_ref[...]
    m1 = z * jax.nn.sigmoid(z)

    a = bdot(m1, We2b_ref[...]) + be2t_ref[...]
    m2 = a * jax.nn.sigmoid(a)
    att = jax.nn.sigmoid(bdot(m2, Wab_ref[...]) + ba_ref[...])
    attb = bdot(att, G4_ref[...])
    m3 = m2 * attb

    cpre = bdot(m3, Wc1b_ref[...]) + bc1t_ref[...]
    c = cpre * jax.nn.sigmoid(cpre)
    cw = jnp.tanh(bdot(c, Wc2b_ref[...]))
    w = cw / (d + 1.0) * wmask4_ref[...]
    prod = jnp.concatenate([dx * w, dy * w, dz * w, jnp.zeros((RP, 4), F32)],
                           axis=1)
    U = f32(S2f_ref[...], prod)
    upd = f32(U, F16_ref[...])
    xo_ref[bb] = xn[:NP] + CR * upd

    m3m = m3 * maskp_ref[...]
    mi128 = jnp.dot(S2_ref[...], m3m.astype(BF),
                    preferred_element_type=F32)
    m_i = bdot(mi128, F4_ref[...])
    hm = jnp.concatenate([h64[:NP], m_i], axis=1)
    t = bdot(hm, Wn1_ref[...]) + bn1_ref[...]
    t = t * jax.nn.sigmoid(t)
    hu = bdot(t, Wn2_ref[...]) + bn2_ref[...]
    ho_ref[bb] = h64[:NP] + hu


def kernel(x, h, d_static, We1, be1, We2, be2, Wn1, bn1, Wn2, bn2,
           Wc1, bc1, Wc2, Wa, ba):
    B = x.shape[0]
    xv = x.reshape(B, NP, 3)
    xn = jnp.pad(xv, ((0, 0), (0, NJ - NP), (0, 5)))
    xjB = jnp.pad(
        jnp.pad(xv, ((0, 0), (0, NJ - NP), (0, 0)))
        .reshape(B, NQ, 4, 3).transpose(0, 1, 3, 2).reshape(B, NQ, 12),
        ((0, 0), (0, 0), (0, 4)))
    hp = jnp.pad(h, ((0, 0), (0, NJ - NP), (0, 0)))
    hB = hp.reshape(B, NQ, 4, NF).reshape(B, NQ, 4 * NF)
    dsp = jnp.pad(d_static, ((0, 0), (0, 0), (0, NJ - (NP - 1))))
    dss = jnp.concatenate(
        [jnp.zeros((B, NP, 1), F32), dsp[:, :, :NJ - 1]], axis=2)
    dsbP = dsp.reshape(B, RP, 4)
    dsSP = dss.reshape(B, RP, 4)

    eye4 = jnp.eye(4, dtype=F32)
    Wh1 = We1[:NF]
    Wh2 = We1[NF:2 * NF]
    wd = We1[2 * NF:2 * NF + 1]
    ws = We1[2 * NF + 1:2 * NF + 2]
    consts = dict(
        R16=jnp.asarray(_R16, BF), R16f=jnp.asarray(_R16),
        R55=jnp.asarray(_R55), S2=jnp.asarray(_S2, BF),
        S2f=jnp.asarray(_S2), maskp=jnp.asarray(_MASKP),
        wmask4=jnp.asarray(_WMASK4), selmask=jnp.asarray(_SELMASK),
        F16=jnp.asarray(_F16),
        Wh1t=jnp.tile(Wh1, (1, 4)).astype(BF),
        Wh2b=jnp.kron(eye4, Wh2).astype(BF),
        wdb=jnp.kron(eye4, wd).astype(BF),
        wsb=jnp.kron(eye4, ws).astype(BF),
        be1t=jnp.tile(be1.reshape(1, NH), (1, 4)),
        We2b=jnp.kron(eye4, We2).astype(BF),
        be2t=jnp.tile(be2.reshape(1, NF), (1, 4)),
        Wab=jnp.kron(eye4, Wa).astype(BF),
        ba=ba.reshape(1, 1),
        G4=jnp.kron(eye4, jnp.ones((1, NF), F32)).astype(BF),
        Wc1b=jnp.kron(eye4, Wc1).astype(BF),
        bc1t=jnp.tile(bc1.reshape(1, NH), (1, 4)),
        Wc2b=jnp.kron(eye4, Wc2).astype(BF),
        F4=jnp.tile(jnp.eye(NF, dtype=F32), (4, 1)).astype(BF),
        Wn1=Wn1.astype(BF), bn1=bn1.reshape(1, NH),
        Wn2=Wn2.astype(BF), bn2=bn2.reshape(1, NF),
    )

    def batch_spec(shp):
        return pl.BlockSpec((BB,) + shp, lambda b: (b, 0, 0))

    def const_spec(arr):
        return pl.BlockSpec(arr.shape, lambda b: (0, 0))

    batch_args = (xn, xjB, hp, hB, dsbP, dsSP)
    batch_shapes = ((NJ, 8), (NQ, 16), (NJ, NF), (NQ, 4 * NF), (RP, 4), (RP, 4))
    const_args = tuple(consts.values())

    in_specs = ([batch_spec(s) for s in batch_shapes]
                + [const_spec(a) for a in const_args])
    out_specs = (batch_spec((NP, 8)), batch_spec((NP, NF)))
    out_shape = (jax.ShapeDtypeStruct((B, NP, 8), F32),
                 jax.ShapeDtypeStruct((B, NP, NF), F32))

    xo, ho = pl.pallas_call(
        _body, grid=(B // BB,), in_specs=in_specs, out_specs=out_specs,
        out_shape=out_shape,
        compiler_params=pltpu.CompilerParams(
            dimension_semantics=("parallel",)))(*batch_args, *const_args)
    return xo[:, :, :3], ho


# speedup vs baseline: 2.5963x; 1.1831x over previous
import jax
import jax.numpy as jnp
import numpy as np
from jax.experimental import pallas as pl
from jax.experimental.pallas import tpu as pltpu

NP = 55
NJ = 56
NQ = NJ // 4
NQP = 16
RP = NP * NQ
NF = 32
NH = 32
CR = 5.0
BB = 4
RT = BB * RP
BF = jnp.bfloat16
F32 = jnp.float32


def _consts():
    R16 = np.zeros((RT, BB * NJ), np.float32)
    R55 = np.zeros((RT, BB * NQP), np.float32)
    S2 = np.zeros((BB * NP, RT), np.float32)
    maskp = np.zeros((RT, 4 * NF), np.float32)
    wmask4 = np.zeros((RT, 4), np.float32)
    selmask = np.zeros((RT, 4), np.float32)
    for bb in range(BB):
        for i in range(NP):
            for q in range(NQ):
                r = bb * RP + i * NQ + q
                R16[r, bb * NJ + i] = 1.0
                R55[r, bb * NQP + q] = 1.0
                S2[bb * NP + i, r] = 1.0
                for g in range(4):
                    j = 4 * q + g
                    valid = (j != i) and (j < NP)
                    if valid:
                        maskp[r, NF * g:NF * (g + 1)] = 1.0
                        wmask4[r, g] = 1.0
                    if j < i:
                        selmask[r, g] = 1.0
    F16 = np.zeros((16, 8), np.float32)
    for d in range(3):
        for g in range(4):
            F16[4 * d + g, d] = 1.0
    return R16, R55, S2, maskp, wmask4, selmask, F16


_R16, _R55, _S2, _MASKP, _WMASK4, _SELMASK, _F16 = _consts()


def _body(xi16_ref, xj16_ref, xn55_ref, h55_ref, hp_ref, hB_ref,
          dsbP_ref, dsSP_ref,
          R16_ref, R55_ref, S2_ref, maskp_ref,
          wmask4_ref, selmask_ref, F16_ref,
          Wh1t_ref, Wh2b_ref, wdb_ref, wsb_ref, be1t_ref,
          We2b_ref, be2t_ref, Wab_ref, ba_ref, G4_ref,
          Wc1b_ref, bc1t_ref, Wc2b_ref, F4_ref,
          Wn1_ref, bn1_ref, Wn2_ref, bn2_ref,
          xo_ref, ho_ref):
    f32 = lambda a, b: jnp.dot(a, b, preferred_element_type=F32)
    bdot = lambda a, b: jnp.dot(a.astype(BF), b, preferred_element_type=F32)

    dvec = xi16_ref[0] - xj16_ref[0]
    sq = dvec * dvec
    d2 = sq[:, 0:4] + sq[:, 4:8] + sq[:, 8:12] + 1e-6
    d = jnp.sqrt(d2)

    sel = selmask_ref[...]
    dsf = dsSP_ref[0] + sel * (dsbP_ref[0] - dsSP_ref[0])
    ds2 = dsf * dsf

    Ptile = bdot(hp_ref[0], Wh1t_ref[...])
    Ppack = jnp.dot(R16_ref[...], Ptile.astype(BF),
                    preferred_element_type=F32)
    QB = bdot(hB_ref[0], Wh2b_ref[...])
    Qpack = jnp.dot(R55_ref[...], QB.astype(BF),
                    preferred_element_type=F32)
    z = (Ppack + Qpack + bdot(d2, wdb_ref[...]) + bdot(ds2, wsb_ref[...])
         + be1t_ref[...])
    m1 = z * jax.nn.sigmoid(z)

    a = bdot(m1, We2b_ref[...]) + be2t_ref[...]
    m2 = a * jax.nn.sigmoid(a)
    att = jax.nn.sigmoid(bdot(m2, Wab_ref[...]) + ba_ref[...])
    attb = bdot(att, G4_ref[...])
    m3 = m2 * attb

    cpre = bdot(m3, Wc1b_ref[...]) + bc1t_ref[...]
    c = cpre * jax.nn.sigmoid(cpre)
    cw = jnp.tanh(bdot(c, Wc2b_ref[...]))
    w = cw / (d + 1.0) * wmask4_ref[...]
    w16 = jnp.concatenate([w, w, w, w], axis=1)
    prod = dvec * w16
    U = jnp.dot(S2_ref[...], prod.astype(BF),
                preferred_element_type=F32)
    upd = f32(U, F16_ref[...])
    xo_ref[0] = xn55_ref[0] + CR * upd

    m3m = m3 * maskp_ref[...]
    mi128 = jnp.dot(S2_ref[...], m3m.astype(BF),
                    preferred_element_type=F32)
    m_i = bdot(mi128, F4_ref[...])
    hm = jnp.concatenate([h55_ref[0], m_i], axis=1)
    t = bdot(hm, Wn1_ref[...]) + bn1_ref[...]
    t = t * jax.nn.sigmoid(t)
    hu = bdot(t, Wn2_ref[...]) + bn2_ref[...]
    ho_ref[0] = h55_ref[0] + hu


def kernel(x, h, d_static, We1, be1, We2, be2, Wn1, bn1, Wn2, bn2,
           Wc1, bc1, Wc2, Wa, ba):
    B = x.shape[0]
    G = B // BB
    xv = x.reshape(B, NP, 3)
    xvp = jnp.pad(xv, ((0, 0), (0, NJ - NP), (0, 0)))

    xi12 = jnp.repeat(xv, 4, axis=2)
    xi16 = jnp.pad(
        jnp.broadcast_to(xi12[:, :, None, :], (B, NP, NQ, 12))
        .reshape(B, RP, 12), ((0, 0), (0, 0), (0, 4))).reshape(G, RT, 16)
    xj12 = (xvp.reshape(B, NQ, 4, 3).transpose(0, 1, 3, 2)
            .reshape(B, 1, NQ, 12))
    xj16 = jnp.pad(
        jnp.broadcast_to(xj12, (B, NP, NQ, 12)).reshape(B, RP, 12),
        ((0, 0), (0, 0), (0, 4))).reshape(G, RT, 16)

    xn55 = jnp.pad(xv, ((0, 0), (0, 0), (0, 5))).reshape(G, BB * NP, 8)
    h55 = h.reshape(G, BB * NP, NF)
    hp = jnp.pad(h, ((0, 0), (0, NJ - NP), (0, 0))).reshape(G, BB * NJ, NF)
    hB = jnp.pad(
        jnp.pad(h, ((0, 0), (0, NJ - NP), (0, 0)))
        .reshape(B, NQ, 4 * NF), ((0, 0), (0, NQP - NQ), (0, 0))
    ).reshape(G, BB * NQP, 4 * NF)
    dsp = jnp.pad(d_static, ((0, 0), (0, 0), (0, NJ - (NP - 1))))
    dss = jnp.concatenate(
        [jnp.zeros((B, NP, 1), F32), dsp[:, :, :NJ - 1]], axis=2)
    dsbP = dsp.reshape(G, RT, 4)
    dsSP = dss.reshape(G, RT, 4)

    eye4 = jnp.eye(4, dtype=F32)
    Wh1 = We1[:NF]
    Wh2 = We1[NF:2 * NF]
    wd = We1[2 * NF:2 * NF + 1]
    ws = We1[2 * NF + 1:2 * NF + 2]
    consts = dict(
        R16=jnp.asarray(_R16, BF), R55=jnp.asarray(_R55, BF),
        S2=jnp.asarray(_S2, BF), maskp=jnp.asarray(_MASKP),
        wmask4=jnp.asarray(_WMASK4), selmask=jnp.asarray(_SELMASK),
        F16=jnp.asarray(_F16),
        Wh1t=jnp.tile(Wh1, (1, 4)).astype(BF),
        Wh2b=jnp.kron(eye4, Wh2).astype(BF),
        wdb=jnp.kron(eye4, wd).astype(BF),
        wsb=jnp.kron(eye4, ws).astype(BF),
        be1t=jnp.tile(be1.reshape(1, NH), (1, 4)),
        We2b=jnp.kron(eye4, We2).astype(BF),
        be2t=jnp.tile(be2.reshape(1, NF), (1, 4)),
        Wab=jnp.kron(eye4, Wa).astype(BF),
        ba=ba.reshape(1, 1),
        G4=jnp.kron(eye4, jnp.ones((1, NF), F32)).astype(BF),
        Wc1b=jnp.kron(eye4, Wc1).astype(BF),
        bc1t=jnp.tile(bc1.reshape(1, NH), (1, 4)),
        Wc2b=jnp.kron(eye4, Wc2).astype(BF),
        F4=jnp.tile(jnp.eye(NF, dtype=F32), (4, 1)).astype(BF),
        Wn1=Wn1.astype(BF), bn1=bn1.reshape(1, NH),
        Wn2=Wn2.astype(BF), bn2=bn2.reshape(1, NF),
    )

    def batch_spec(shp):
        return pl.BlockSpec((1,) + shp, lambda b: (b, 0, 0))

    def const_spec(arr):
        return pl.BlockSpec(arr.shape, lambda b: (0, 0))

    batch_args = (xi16, xj16, xn55, h55, hp, hB, dsbP, dsSP)
    batch_shapes = ((RT, 16), (RT, 16), (BB * NP, 8), (BB * NP, NF),
                    (BB * NJ, NF), (BB * NQP, 4 * NF), (RT, 4), (RT, 4))
    const_args = tuple(consts.values())

    in_specs = ([batch_spec(s) for s in batch_shapes]
                + [const_spec(a) for a in const_args])
    out_specs = (batch_spec((BB * NP, 8)), batch_spec((BB * NP, NF)))
    out_shape = (jax.ShapeDtypeStruct((G, BB * NP, 8), F32),
                 jax.ShapeDtypeStruct((G, BB * NP, NF), F32))

    xo, ho = pl.pallas_call(
        _body, grid=(G,), in_specs=in_specs, out_specs=out_specs,
        out_shape=out_shape,
        compiler_params=pltpu.CompilerParams(
            dimension_semantics=("parallel",)))(*batch_args, *const_args)
    return (xo.reshape(B, NP, 8)[:, :, :3], ho.reshape(B, NP, NF))


# speedup vs baseline: 3.0159x; 1.1616x over previous
import jax
import jax.numpy as jnp
import numpy as np
from jax.experimental import pallas as pl
from jax.experimental.pallas import tpu as pltpu

NP = 55
NJ = 56
NQ = NJ // 4
NQP = 16
RP = NP * NQ
NF = 32
NH = 32
CR = 5.0
BB = 4
RT = BB * RP
BF = jnp.bfloat16
F32 = jnp.float32


def _consts():
    R16 = np.zeros((RT, BB * NJ), np.float32)
    R55 = np.zeros((RT, BB * NQP), np.float32)
    S2 = np.zeros((BB * NP, RT), np.float32)
    maskp = np.zeros((RT, 4 * NF), np.float32)
    wmask4 = np.zeros((RT, 4), np.float32)
    selmask = np.zeros((RT, 4), np.float32)
    for bb in range(BB):
        for i in range(NP):
            for q in range(NQ):
                r = bb * RP + i * NQ + q
                R16[r, bb * NJ + i] = 1.0
                R55[r, bb * NQP + q] = 1.0
                S2[bb * NP + i, r] = 1.0
                for g in range(4):
                    j = 4 * q + g
                    valid = (j != i) and (j < NP)
                    if valid:
                        maskp[r, NF * g:NF * (g + 1)] = 1.0
                        wmask4[r, g] = 1.0
                    if j < i:
                        selmask[r, g] = 1.0
    F16 = np.zeros((16, 8), np.float32)
    for d in range(3):
        for g in range(4):
            F16[4 * d + g, d] = 1.0
    return R16, R55, S2, maskp, wmask4, selmask, F16


_R16, _R55, _S2, _MASKP, _WMASK4, _SELMASK, _F16 = _consts()


def _body(xi16_ref, xj16_ref, xn55_ref, h55_ref, hp_ref, hB_ref,
          dsbP_ref, dsSP_ref,
          R16_ref, R55_ref, S2_ref, maskp_ref,
          wmask4_ref, selmask_ref, F16_ref,
          Wh1t_ref, Wh2b_ref, W12_ref,
          We2b_ref, be2t_ref, Wab_ref, ba_ref, G4_ref,
          Wc1b_ref, bc1t_ref, Wc2b_ref, F4_ref,
          Wn1_ref, bn1_ref, Wn2_ref, bn2_ref,
          xo_ref, ho_ref):
    fdot = lambda a, b: jnp.dot(a, b, preferred_element_type=F32)
    LOG2E = 1.4426950408889634

    def _sig(v):
        one = jnp.asarray(1.0, v.dtype)
        l2e = jnp.asarray(-LOG2E, v.dtype)
        return one / (one + jnp.exp2(v * l2e))

    def _silu(v):
        return v * _sig(v)

    dvec = xi16_ref[0] - xj16_ref[0]
    sq = dvec * dvec
    d2 = sq[:, 0:4] + sq[:, 4:8] + sq[:, 8:12] + jnp.asarray(1e-6, BF)
    d = jnp.sqrt(d2)

    sel = selmask_ref[...]
    dsf = dsSP_ref[0] + sel * (dsbP_ref[0] - dsSP_ref[0])
    ds2 = dsf * dsf
    gf = jnp.concatenate([d2, ds2, jnp.ones((RT, 4), BF)], axis=1)

    Ptile = fdot(hp_ref[0], Wh1t_ref[...])
    Ppack = fdot(R16_ref[...], Ptile.astype(BF))
    QB = fdot(hB_ref[0], Wh2b_ref[...])
    Qpack = fdot(R55_ref[...], QB.astype(BF))
    z = Ppack + Qpack + fdot(gf, W12_ref[...])
    m1 = _silu(z.astype(BF))

    a = fdot(m1, We2b_ref[...]) + be2t_ref[...]
    m2 = _silu(a.astype(BF))
    att = _sig((fdot(m2, Wab_ref[...]) + ba_ref[...]).astype(BF))
    attb = fdot(att, G4_ref[...])
    m3 = m2 * attb.astype(BF)

    cpre = fdot(m3, Wc1b_ref[...]) + bc1t_ref[...]
    c = _silu(cpre.astype(BF))
    cw = jnp.tanh(fdot(c, Wc2b_ref[...]).astype(BF))
    one = jnp.asarray(1.0, BF)
    wb = cw / (d + one) * wmask4_ref[...]
    w16 = jnp.concatenate([wb, wb, wb, wb], axis=1)
    prod = dvec * w16

    m3m = m3 * maskp_ref[...]
    agg_in = jnp.concatenate([m3m, prod], axis=1)
    AGG = fdot(S2_ref[...], agg_in)
    mi128 = AGG[:, :4 * NF]
    U = AGG[:, 4 * NF:]
    upd = fdot(U, F16_ref[...])
    xo_ref[0] = xn55_ref[0] + CR * upd

    m_i = fdot(mi128.astype(BF), F4_ref[...])
    hm = jnp.concatenate([h55_ref[0], m_i], axis=1)
    t = fdot(hm.astype(BF), Wn1_ref[...]) + bn1_ref[...]
    t = _silu(t)
    hu = fdot(t.astype(BF), Wn2_ref[...]) + bn2_ref[...]
    ho_ref[0] = h55_ref[0] + hu


def kernel(x, h, d_static, We1, be1, We2, be2, Wn1, bn1, Wn2, bn2,
           Wc1, bc1, Wc2, Wa, ba):
    B = x.shape[0]
    G = B // BB
    xv = x.reshape(B, NP, 3)
    xvp = jnp.pad(xv, ((0, 0), (0, NJ - NP), (0, 0)))

    xi12 = jnp.repeat(xv, 4, axis=2)
    xi16 = jnp.pad(
        jnp.broadcast_to(xi12[:, :, None, :], (B, NP, NQ, 12))
        .reshape(B, RP, 12), ((0, 0), (0, 0), (0, 4))).reshape(G, RT, 16)
    xj12 = (xvp.reshape(B, NQ, 4, 3).transpose(0, 1, 3, 2)
            .reshape(B, 1, NQ, 12))
    xj16 = jnp.pad(
        jnp.broadcast_to(xj12, (B, NP, NQ, 12)).reshape(B, RP, 12),
        ((0, 0), (0, 0), (0, 4))).reshape(G, RT, 16)

    xn55 = jnp.pad(xv, ((0, 0), (0, 0), (0, 5))).reshape(G, BB * NP, 8)
    h55 = h.reshape(G, BB * NP, NF)
    hp = jnp.pad(h, ((0, 0), (0, NJ - NP), (0, 0))).reshape(G, BB * NJ, NF)
    hB = jnp.pad(
        jnp.pad(h, ((0, 0), (0, NJ - NP), (0, 0)))
        .reshape(B, NQ, 4 * NF), ((0, 0), (0, NQP - NQ), (0, 0))
    ).reshape(G, BB * NQP, 4 * NF)
    dsp = jnp.pad(d_static, ((0, 0), (0, 0), (0, NJ - (NP - 1))))
    dss = jnp.concatenate(
        [jnp.zeros((B, NP, 1), F32), dsp[:, :, :NJ - 1]], axis=2)
    dsbP = dsp.reshape(G, RT, 4).astype(BF)
    dsSP = dss.reshape(G, RT, 4).astype(BF)

    eye4 = jnp.eye(4, dtype=F32)
    Wh1 = We1[:NF]
    Wh2 = We1[NF:2 * NF]
    wd = We1[2 * NF:2 * NF + 1]
    ws = We1[2 * NF + 1:2 * NF + 2]
    W12 = jnp.concatenate(
        [jnp.kron(eye4, wd), jnp.kron(eye4, ws),
         jnp.tile(be1.reshape(1, NH), (1, 4)),
         jnp.zeros((3, 4 * NH), F32)], axis=0)
    consts = dict(
        R16=jnp.asarray(_R16, BF), R55=jnp.asarray(_R55, BF),
        S2=jnp.asarray(_S2, BF), maskp=jnp.asarray(_MASKP, BF),
        wmask4=jnp.asarray(_WMASK4, BF), selmask=jnp.asarray(_SELMASK, BF),
        F16=jnp.asarray(_F16),
        Wh1t=jnp.tile(Wh1, (1, 4)).astype(BF),
        Wh2b=jnp.kron(eye4, Wh2).astype(BF),
        W12=W12.astype(BF),
        We2b=jnp.kron(eye4, We2).astype(BF),
        be2t=jnp.tile(be2.reshape(1, NF), (1, 4)),
        Wab=jnp.kron(eye4, Wa).astype(BF),
        ba=ba.reshape(1, 1),
        G4=jnp.kron(eye4, jnp.ones((1, NF), F32)).astype(BF),
        Wc1b=jnp.kron(eye4, Wc1).astype(BF),
        bc1t=jnp.tile(bc1.reshape(1, NH), (1, 4)),
        Wc2b=jnp.kron(eye4, Wc2).astype(BF),
        F4=jnp.tile(jnp.eye(NF, dtype=F32), (4, 1)).astype(BF),
        Wn1=Wn1.astype(BF), bn1=bn1.reshape(1, NH),
        Wn2=Wn2.astype(BF), bn2=bn2.reshape(1, NF),
    )

    def batch_spec(shp):
        return pl.BlockSpec((1,) + shp, lambda b: (b, 0, 0))

    def const_spec(arr):
        return pl.BlockSpec(arr.shape, lambda b: (0, 0))

    batch_args = (xi16.astype(BF), xj16.astype(BF), xn55, h55,
                  hp.astype(BF), hB.astype(BF), dsbP, dsSP)
    batch_shapes = ((RT, 16), (RT, 16), (BB * NP, 8), (BB * NP, NF),
                    (BB * NJ, NF), (BB * NQP, 4 * NF), (RT, 4), (RT, 4))
    const_args = tuple(consts.values())

    in_specs = ([batch_spec(s) for s in batch_shapes]
                + [const_spec(a) for a in const_args])
    out_specs = (batch_spec((BB * NP, 8)), batch_spec((BB * NP, NF)))
    out_shape = (jax.ShapeDtypeStruct((G, BB * NP, 8), F32),
                 jax.ShapeDtypeStruct((G, BB * NP, NF), F32))

    xo, ho = pl.pallas_call(
        _body, grid=(G,), in_specs=in_specs, out_specs=out_specs,
        out_shape=out_shape,
        compiler_params=pltpu.CompilerParams(
            dimension_semantics=("parallel",)))(*batch_args, *const_args)
    return (xo.reshape(B, NP, 8)[:, :, :3], ho.reshape(B, NP, NF))
